# baseline (speedup 1.0000x reference)
import jax
import jax.numpy as jnp
from jax import lax
from jax.experimental import pallas as pl
from jax.experimental.pallas import tpu as pltpu
from jax.experimental.pallas import tpu_sc as plsc

_W = 640
_RPI = 1920
_EPI = _RPI // 8
_EPW = _EPI // 2
_TC = _W // 128
_GE = 4
_NT = _GE * _TC
_NG = _EPW // _GE
_NGH = _NG // 2
_LANES = 16


def _sc_body(img_hbm, mask_hbm, rc_hbm, ri_hbm, out_hbm,
             ibuf0, ibuf1, mbuf0, mbuf1, obuf0, obuf1, rc_v, ri_v, eidx_v,
             i_sem0, i_sem1, m_sem0, m_sem1, o_sem0, o_sem1):
    half = lax.axis_index("c")
    b = lax.axis_index("s")

    pltpu.sync_copy(rc_hbm, rc_v)
    pltpu.sync_copy(ri_hbm, ri_v)
    rc = rc_v[...]
    ri = ri_v[...]
    catf = jnp.where(rc <= 0.001, 0.0, 1.0)
    catf = jnp.where(rc > 0.5, 2.0, catf)
    x = (catf - 1.0) * 10.0 + ri * 10.0
    t = x.astype(jnp.int32)
    idx = t + jnp.where(x > t.astype(jnp.float32), 1, 0)
    idx = jnp.clip(idx, 0, 20)
    lanes = lax.iota(jnp.int32, _LANES)
    bvec = jnp.full((_LANES,), b, jnp.int32)
    i_b_vec = lax.gather(
        idx, bvec[:, None],
        lax.GatherDimensionNumbers(offset_dims=(), collapsed_slice_dims=(0,),
                                   start_index_map=(0,)),
        slice_sizes=(1,), mode=lax.GatherScatterMode.PROMISE_IN_BOUNDS)

    ebase_vec = i_b_vec * _EPI + half * _EPW
    def fill(v, _):
        p = 16 * v + lanes
        g = p >> 3
        j = jnp.minimum(p & 7, 3)
        eidx_v[pl.ds(16 * v, 16)] = ebase_vec + g * _GE + j
        return 0
    lax.fori_loop(0, (_NG * 8) // 16, fill, 0)

    irow = b * _RPI + half * (_RPI // 2)

    ibufs = (ibuf0, ibuf1)
    mbufs = (mbuf0, mbuf1)
    obufs = (obuf0, obuf1)
    isems = (i_sem0, i_sem1)
    msems = (m_sem0, m_sem1)
    osems = (o_sem0, o_sem1)

    def issue_in(g, slot):
        base = irow + g * (_GE * 8)
        for tr in range(_GE):
            for tc in range(_TC):
                pltpu.async_copy(
                    img_hbm.at[pl.ds(base + tr * 8, 8),
                               pl.ds(tc * 128, 128)],
                    ibufs[slot].at[tr * _TC + tc], isems[slot])
        for tc in range(_TC):
            pltpu.async_copy(
                mask_hbm.at[eidx_v.at[pl.ds(8 * g, _GE)], slice(None),
                            pl.ds(tc * 128, 128)],
                mbufs[slot].at[tc], msems[slot])

    issue_in(0, 0)
    issue_in(1, 1)

    def halfstep(it, slot):
        g = 2 * it + slot

        @pl.when(it > 0)
        def _():
            for t in range(_NT):
                pltpu.make_async_copy(
                    obufs[slot].at[0], out_hbm.at[pl.ds(0, 8), pl.ds(0, 128)],
                    osems[slot]).wait()

        for t in range(_NT):
            pltpu.make_async_copy(
                img_hbm.at[pl.ds(0, 8), pl.ds(0, 128)],
                ibufs[slot].at[0], isems[slot]).wait()
        for tc in range(_TC):
            pltpu.make_async_copy(
                mask_hbm.at[eidx_v.at[pl.ds(0, _GE)], slice(None),
                            pl.ds(0, 128)],
                mbufs[slot].at[0], msems[slot]).wait()

        ib, mb, ob = ibufs[slot], mbufs[slot], obufs[slot]

        def entry(tr, _):
            for tc in range(_TC):
                for r in range(8):
                    for c in range(8):
                        sl = pl.ds(c * _LANES, _LANES)
                        m = mb[tc, tr, r, sl]
                        ob[tr * _TC + tc, r, sl] = (
                            ib[tr * _TC + tc, r, sl] * (1.0 - m) + m)
            return 0
        lax.fori_loop(0, _GE, entry, 0)

        base = irow + g * (_GE * 8)
        for tr in range(_GE):
            for tc in range(_TC):
                pltpu.async_copy(
                    obufs[slot].at[tr * _TC + tc],
                    out_hbm.at[pl.ds(base + tr * 8, 8), pl.ds(tc * 128, 128)],
                    osems[slot])

        @pl.when(it < _NGH - 1)
        def _():
            issue_in(g + 2, slot)

    def body(it, _):
        halfstep(it, 0)
        halfstep(it, 1)
        return 0
    lax.fori_loop(0, _NGH, body, 0)

    for slot in (0, 1):
        for t in range(_NT):
            pltpu.make_async_copy(
                obufs[slot].at[0], out_hbm.at[pl.ds(0, 8), pl.ds(0, 128)],
                osems[slot]).wait()


def kernel(img_batch, Mask, rand_category, rand_index):
    B, C, H, W = img_batch.shape
    img2 = img_batch.reshape(B * _RPI, _W)
    mask3 = Mask.reshape(Mask.shape[0] * _EPI, 8, _W)
    mesh = plsc.VectorSubcoreMesh(core_axis_name="c", subcore_axis_name="s")
    kfn = pl.kernel(
        _sc_body,
        out_type=jax.ShapeDtypeStruct((B * _RPI, _W), jnp.float32),
        mesh=mesh,
        scratch_types=[
            pltpu.VMEM((_NT, 8, 128), jnp.float32),
            pltpu.VMEM((_NT, 8, 128), jnp.float32),
            pltpu.VMEM((_TC, _GE, 8, 128), jnp.float32),
            pltpu.VMEM((_TC, _GE, 8, 128), jnp.float32),
            pltpu.VMEM((_NT, 8, 128), jnp.float32),
            pltpu.VMEM((_NT, 8, 128), jnp.float32),
            pltpu.VMEM((_LANES,), jnp.float32),
            pltpu.VMEM((_LANES,), jnp.float32),
            pltpu.VMEM((_NG * 8,), jnp.int32),
            pltpu.SemaphoreType.DMA,
            pltpu.SemaphoreType.DMA,
            pltpu.SemaphoreType.DMA,
            pltpu.SemaphoreType.DMA,
            pltpu.SemaphoreType.DMA,
            pltpu.SemaphoreType.DMA,
        ],
    )
    out = kfn(img2, mask3, rand_category, rand_index)
    return out.reshape(B, C, H, W)


# speedup vs baseline: 5.1829x; 5.1829x over previous
import jax
import jax.numpy as jnp
from jax import lax
from jax.experimental import pallas as pl
from jax.experimental.pallas import tpu as pltpu
from jax.experimental.pallas import tpu_sc as plsc

_W = 640
_RPI = 1920
_EPI = _RPI // 8
_EPW = _EPI // 2
_TC = _W // 128
_GE = 4
_NT = _GE * _TC
_NG = _EPW // _GE
_NGH = _NG // 2
_LANES = 16


def _sc_body(img_hbm, mask_hbm, rc_hbm, ri_hbm, out_hbm,
             ibuf0, ibuf1, mbuf0, mbuf1, obuf0, obuf1, rc_v, ri_v, eidx_v,
             i_sem0, i_sem1, m_sem0, m_sem1, o_sem0, o_sem1):
    half = lax.axis_index("c")
    b = lax.axis_index("s")

    pltpu.sync_copy(rc_hbm, rc_v)
    pltpu.sync_copy(ri_hbm, ri_v)
    rc = rc_v[...]
    ri = ri_v[...]
    catf = jnp.where(rc <= 0.001, 0.0, 1.0)
    catf = jnp.where(rc > 0.5, 2.0, catf)
    x = (catf - 1.0) * 10.0 + ri * 10.0
    t = x.astype(jnp.int32)
    idx = t + jnp.where(x > t.astype(jnp.float32), 1, 0)
    idx = jnp.clip(idx, 0, 20)
    lanes = lax.iota(jnp.int32, _LANES)
    bvec = jnp.full((_LANES,), b, jnp.int32)
    i_b_vec = lax.gather(
        idx, bvec[:, None],
        lax.GatherDimensionNumbers(offset_dims=(), collapsed_slice_dims=(0,),
                                   start_index_map=(0,)),
        slice_sizes=(1,), mode=lax.GatherScatterMode.PROMISE_IN_BOUNDS)

    ebase_vec = i_b_vec * _EPI + half * _EPW
    def fill(v, _):
        p = 16 * v + lanes
        g = p >> 3
        j = jnp.minimum(p & 7, 3)
        eidx_v[pl.ds(16 * v, 16)] = ebase_vec + g * _GE + j
        return 0
    lax.fori_loop(0, (_NG * 8) // 16, fill, 0)

    irow = b * _RPI + half * (_RPI // 2)

    ibufs = (ibuf0, ibuf1)
    mbufs = (mbuf0, mbuf1)
    obufs = (obuf0, obuf1)
    isems = (i_sem0, i_sem1)
    msems = (m_sem0, m_sem1)
    osems = (o_sem0, o_sem1)

    def issue_in(g, slot):
        base = irow + g * (_GE * 8)
        for tr in range(_GE):
            for tc in range(_TC):
                pltpu.async_copy(
                    img_hbm.at[pl.ds(base + tr * 8, 8),
                               pl.ds(tc * 128, 128)],
                    ibufs[slot].at[tr * _TC + tc], isems[slot])
        for tc in range(_TC):
            pltpu.async_copy(
                mask_hbm.at[eidx_v.at[pl.ds(8 * g, _GE)], slice(None),
                            pl.ds(tc * 128, 128)],
                mbufs[slot].at[tc], msems[slot])

    issue_in(0, 0)
    issue_in(1, 1)

    def halfstep(it, slot):
        g = 2 * it + slot

        @pl.when(it > 0)
        def _():
            for t in range(_NT):
                pltpu.make_async_copy(
                    obufs[slot].at[0], out_hbm.at[pl.ds(0, 8), pl.ds(0, 128)],
                    osems[slot]).wait()

        for t in range(_NT):
            pltpu.make_async_copy(
                img_hbm.at[pl.ds(0, 8), pl.ds(0, 128)],
                ibufs[slot].at[0], isems[slot]).wait()
        for tc in range(_TC):
            pltpu.make_async_copy(
                mask_hbm.at[eidx_v.at[pl.ds(0, _GE)], slice(None),
                            pl.ds(0, 128)],
                mbufs[slot].at[0], msems[slot]).wait()

        ib, mb, ob = ibufs[slot], mbufs[slot], obufs[slot]

        ib, mb, ob = ibufs[slot], mbufs[slot], obufs[slot]

        def chunk(ci, _):
            r = ci >> 3
            cs = (ci & 7) * _LANES
            sl = pl.ds(cs, _LANES)
            for tr in range(_GE):
                for tc in range(_TC):
                    m = mb[tc, tr, r, sl]
                    ob[tr * _TC + tc, r, sl] = (
                        ib[tr * _TC + tc, r, sl] * (1.0 - m) + m)
            return 0
        lax.fori_loop(0, 64, chunk, 0)

        base = irow + g * (_GE * 8)
        for tr in range(_GE):
            for tc in range(_TC):
                pltpu.async_copy(
                    obufs[slot].at[tr * _TC + tc],
                    out_hbm.at[pl.ds(base + tr * 8, 8), pl.ds(tc * 128, 128)],
                    osems[slot])

        @pl.when(it < _NGH - 1)
        def _():
            issue_in(g + 2, slot)

    def body(it, _):
        halfstep(it, 0)
        halfstep(it, 1)
        return 0
    lax.fori_loop(0, _NGH, body, 0)

    for slot in (0, 1):
        for t in range(_NT):
            pltpu.make_async_copy(
                obufs[slot].at[0], out_hbm.at[pl.ds(0, 8), pl.ds(0, 128)],
                osems[slot]).wait()


def kernel(img_batch, Mask, rand_category, rand_index):
    B, C, H, W = img_batch.shape
    img2 = img_batch.reshape(B * _RPI, _W)
    mask3 = Mask.reshape(Mask.shape[0] * _EPI, 8, _W)
    mesh = plsc.VectorSubcoreMesh(core_axis_name="c", subcore_axis_name="s")
    kfn = pl.kernel(
        _sc_body,
        out_type=jax.ShapeDtypeStruct((B * _RPI, _W), jnp.float32),
        mesh=mesh,
        scratch_types=[
            pltpu.VMEM((_NT, 8, 128), jnp.float32),
            pltpu.VMEM((_NT, 8, 128), jnp.float32),
            pltpu.VMEM((_TC, _GE, 8, 128), jnp.float32),
            pltpu.VMEM((_TC, _GE, 8, 128), jnp.float32),
            pltpu.VMEM((_NT, 8, 128), jnp.float32),
            pltpu.VMEM((_NT, 8, 128), jnp.float32),
            pltpu.VMEM((_LANES,), jnp.float32),
            pltpu.VMEM((_LANES,), jnp.float32),
            pltpu.VMEM((_NG * 8,), jnp.int32),
            pltpu.SemaphoreType.DMA,
            pltpu.SemaphoreType.DMA,
            pltpu.SemaphoreType.DMA,
            pltpu.SemaphoreType.DMA,
            pltpu.SemaphoreType.DMA,
            pltpu.SemaphoreType.DMA,
        ],
    )
    out = kfn(img2, mask3, rand_category, rand_index)
    return out.reshape(B, C, H, W)


# speedup vs baseline: 5.1940x; 1.0021x over previous
import jax
import jax.numpy as jnp
from jax import lax
from jax.experimental import pallas as pl
from jax.experimental.pallas import tpu as pltpu
from jax.experimental.pallas import tpu_sc as plsc

_W = 640
_RPI = 1920
_EPI = _RPI // 8
_EPW = _EPI // 2
_TC = _W // 128
_GE = 4
_NT = _GE * _TC
_NG = _EPW // _GE
_NGH = _NG // 2
_LANES = 16


def _sc_body(img_hbm, mask_hbm, rc_hbm, ri_hbm, out_hbm,
             ibuf0, ibuf1, mbuf0, mbuf1, obuf0, obuf1, rc_v, ri_v, eidx_v,
             gidx_v, i_sem0, i_sem1, m_sem0, m_sem1, o_sem0, o_sem1):
    half = lax.axis_index("c")
    b = lax.axis_index("s")

    pltpu.sync_copy(rc_hbm, rc_v)
    pltpu.sync_copy(ri_hbm, ri_v)
    rc = rc_v[...]
    ri = ri_v[...]
    catf = jnp.where(rc <= 0.001, 0.0, 1.0)
    catf = jnp.where(rc > 0.5, 2.0, catf)
    x = (catf - 1.0) * 10.0 + ri * 10.0
    t = x.astype(jnp.int32)
    idx = t + jnp.where(x > t.astype(jnp.float32), 1, 0)
    idx = jnp.clip(idx, 0, 20)
    lanes = lax.iota(jnp.int32, _LANES)
    bvec = jnp.full((_LANES,), b, jnp.int32)
    i_b_vec = lax.gather(
        idx, bvec[:, None],
        lax.GatherDimensionNumbers(offset_dims=(), collapsed_slice_dims=(0,),
                                   start_index_map=(0,)),
        slice_sizes=(1,), mode=lax.GatherScatterMode.PROMISE_IN_BOUNDS)

    ebase_vec = i_b_vec * _EPI + half * _EPW
    gbase = b * _EPI + half * _EPW
    def fill(v, _):
        p = 16 * v + lanes
        g = p >> 3
        j = jnp.minimum(p & 7, 3)
        eidx_v[pl.ds(16 * v, 16)] = ebase_vec + g * _GE + j
        gidx_v[pl.ds(16 * v, 16)] = gbase + g * _GE + j
        return 0
    lax.fori_loop(0, (_NG * 8) // 16, fill, 0)

    irow = b * _RPI + half * (_RPI // 2)

    ibufs = (ibuf0, ibuf1)
    mbufs = (mbuf0, mbuf1)
    obufs = (obuf0, obuf1)
    isems = (i_sem0, i_sem1)
    msems = (m_sem0, m_sem1)
    osems = (o_sem0, o_sem1)

    def issue_in(g, slot):
        for tc in range(_TC):
            pltpu.async_copy(
                img_hbm.at[gidx_v.at[pl.ds(8 * g, _GE)], slice(None),
                           pl.ds(tc * 128, 128)],
                ibufs[slot].at[tc], isems[slot])
        for tc in range(_TC):
            pltpu.async_copy(
                mask_hbm.at[eidx_v.at[pl.ds(8 * g, _GE)], slice(None),
                            pl.ds(tc * 128, 128)],
                mbufs[slot].at[tc], msems[slot])

    issue_in(0, 0)
    issue_in(1, 1)

    def halfstep(it, slot):
        g = 2 * it + slot

        @pl.when(it > 0)
        def _():
            for t in range(_NT):
                pltpu.make_async_copy(
                    obufs[slot].at[0], out_hbm.at[pl.ds(0, 8), pl.ds(0, 128)],
                    osems[slot]).wait()

        for tc in range(_TC):
            pltpu.make_async_copy(
                img_hbm.at[gidx_v.at[pl.ds(0, _GE)], slice(None),
                           pl.ds(0, 128)],
                ibufs[slot].at[0], isems[slot]).wait()
        for tc in range(_TC):
            pltpu.make_async_copy(
                mask_hbm.at[eidx_v.at[pl.ds(0, _GE)], slice(None),
                            pl.ds(0, 128)],
                mbufs[slot].at[0], msems[slot]).wait()

        ib, mb, ob = ibufs[slot], mbufs[slot], obufs[slot]

        def chunk(ci, _):
            r = ci >> 3
            cs = (ci & 7) * _LANES
            sl = pl.ds(cs, _LANES)
            for tr in range(_GE):
                for tc in range(_TC):
                    m = mb[tc, tr, r, sl]
                    ob[tr * _TC + tc, r, sl] = (
                        ib[tc, tr, r, sl] * (1.0 - m) + m)
            return 0
        lax.fori_loop(0, 64, chunk, 0)

        base = irow + g * (_GE * 8)
        for tr in range(_GE):
            for tc in range(_TC):
                pltpu.async_copy(
                    obufs[slot].at[tr * _TC + tc],
                    out_hbm.at[pl.ds(base + tr * 8, 8), pl.ds(tc * 128, 128)],
                    osems[slot])

        @pl.when(it < _NGH - 1)
        def _():
            issue_in(g + 2, slot)

    def body(it, _):
        halfstep(it, 0)
        halfstep(it, 1)
        return 0
    lax.fori_loop(0, _NGH, body, 0)

    for slot in (0, 1):
        for t in range(_NT):
            pltpu.make_async_copy(
                obufs[slot].at[0], out_hbm.at[pl.ds(0, 8), pl.ds(0, 128)],
                osems[slot]).wait()


def kernel(img_batch, Mask, rand_category, rand_index):
    B, C, H, W = img_batch.shape
    img3 = img_batch.reshape(B * _EPI, 8, _W)
    mask3 = Mask.reshape(Mask.shape[0] * _EPI, 8, _W)
    mesh = plsc.VectorSubcoreMesh(core_axis_name="c", subcore_axis_name="s")
    kfn = pl.kernel(
        _sc_body,
        out_type=jax.ShapeDtypeStruct((B * _RPI, _W), jnp.float32),
        mesh=mesh,
        scratch_types=[
            pltpu.VMEM((_TC, _GE, 8, 128), jnp.float32),
            pltpu.VMEM((_TC, _GE, 8, 128), jnp.float32),
            pltpu.VMEM((_TC, _GE, 8, 128), jnp.float32),
            pltpu.VMEM((_TC, _GE, 8, 128), jnp.float32),
            pltpu.VMEM((_NT, 8, 128), jnp.float32),
            pltpu.VMEM((_NT, 8, 128), jnp.float32),
            pltpu.VMEM((_LANES,), jnp.float32),
            pltpu.VMEM((_LANES,), jnp.float32),
            pltpu.VMEM((_NG * 8,), jnp.int32),
            pltpu.VMEM((_NG * 8,), jnp.int32),
            pltpu.SemaphoreType.DMA,
            pltpu.SemaphoreType.DMA,
            pltpu.SemaphoreType.DMA,
            pltpu.SemaphoreType.DMA,
            pltpu.SemaphoreType.DMA,
            pltpu.SemaphoreType.DMA,
        ],
    )
    out = kfn(img3, mask3, rand_category, rand_index)
    return out.reshape(B, C, H, W)
